# Initial kernel scaffold; baseline (speedup 1.0000x reference)
import jax
import jax.numpy as jnp
from jax.experimental import pallas as pl


def kernel(positions, cells, species, cell_shifts, centers, pairs, structure_centers, structure_pairs, structure_offsets):
    raise NotImplementedError("write your pallas kernel here")


# speedup vs baseline: 1470.2761x; 1470.2761x over previous
import jax
import jax.numpy as jnp
from jax import lax
from jax.experimental import pallas as pl
from jax.experimental.pallas import tpu as pltpu
from jax.experimental.pallas import tpu_sc as plsc

SPECIES_VALS = (1, 6, 7, 8, 15, 16)
NSP = 6
NA = 1048576
NS = 1024
NC, NSUBC, L = 2, 16, 16
NW = NC * NSUBC
CHUNK = NA // NW
SUB = 512
NSUB_W = CHUNK // SUB
NGRP = NSUB_W // L
NSUB = NW * NSUB_W
OFF_W = NS // NW
GROWS = 48
CP = 48

_MESH = plsc.VectorSubcoreMesh(
    core_axis_name="c", subcore_axis_name="s", num_cores=NC, num_subcores=NSUBC)


def _wid():
  return lax.axis_index("s") * NC + lax.axis_index("c")


def _hist_body(species_hbm, cum_hbm, chunk_v, cum_v, sem):
  wid = _wid()
  pltpu.async_copy(species_hbm.at[pl.ds(wid * CHUNK, CHUNK)], chunk_v, sem).wait()

  lane = lax.iota(jnp.int32, L)
  carries = [jnp.zeros((), jnp.int32) for _ in range(NSP)]
  for g in range(NGRP):
    base_idx = (g * L + lane) * SUB

    def step(t, accs):
      v = plsc.load_gather(chunk_v, [base_idx + t])
      return tuple(a + (v == sv).astype(jnp.int32)
                   for a, sv in zip(accs, SPECIES_VALS))

    accs = lax.fori_loop(
        0, SUB, step, tuple(jnp.zeros((L,), jnp.int32) for _ in range(NSP)))
    for s in range(NSP):
      inc = plsc.cumsum(accs[s]) + carries[s]
      cum_v[pl.ds(s * NSUB_W + g * L, L)] = inc
      carries[s] = carries[s] + jnp.sum(accs[s])

  for s in range(NSP):
    pltpu.sync_copy(cum_v.at[pl.ds(s * NSUB_W, NSUB_W)],
                    cum_hbm.at[pl.ds(s * NSUB + wid * NSUB_W, NSUB_W)])


def _compose_body(species2d_hbm, offsets_hbm, cum_hbm, out_hbm,
                  off_v, cum_v, cpref_v, idx_v, gbuf, outbuf, sem):
  wid = _wid()
  pltpu.async_copy(offsets_hbm, off_v, sem).wait()
  pltpu.async_copy(cum_hbm, cum_v, sem).wait()

  lane = lax.iota(jnp.int32, L)

  for s in range(NSP):
    carry = jnp.zeros((), jnp.int32)
    for g in range(NW // L):
      tot = plsc.load_gather(
          cum_v, [s * NSUB + (g * L + lane) * NSUB_W + (NSUB_W - 1)])
      cpref_v[pl.ds(s * CP + g * L, L)] = plsc.cumsum(tot) - tot + carry
      carry = carry + jnp.sum(tot)
    cpref_v[pl.ds(s * CP + NW, L)] = jnp.broadcast_to(carry, (L,))

  for t in range(GROWS // L):
    ivec = wid * OFF_W + t * L + lane
    ovals = plsc.load_gather(off_v, [jnp.minimum(ivec, NS - 1)])
    idx_v[pl.ds(t * L, L)] = lax.shift_right_logical(ovals, 9)
  pltpu.async_copy(species2d_hbm.at[idx_v], gbuf, sem).wait()

  sp_lane = jnp.minimum(lane, NSP - 1)

  def compute_p(i, r):
    ivec = jnp.broadcast_to(jnp.minimum(i, NS - 1), (L,))
    o = plsc.load_gather(off_v, [ivec])
    o = jnp.where(i >= NS, NA, o)
    ocl = jnp.minimum(o, NA - 1)
    c = lax.shift_right_logical(ocl, 9)
    w = lax.shift_right_logical(c, 6)
    lsub = jnp.bitwise_and(c, NSUB_W - 1)
    rem = jnp.bitwise_and(ocl, SUB - 1)
    nk = lax.shift_right_logical(rem + (L - 1), 4)[0]
    rvec = jnp.broadcast_to(r, (L,))

    def step(k, accs):
      v = plsc.load_gather(gbuf, [rvec, k * L + lane])
      m = (k * L + lane) < rem
      return tuple(a + jnp.logical_and(m, v == sv).astype(jnp.int32)
                   for a, sv in zip(accs, SPECIES_VALS))

    accs = lax.fori_loop(
        0, nk, step, tuple(jnp.zeros((L,), jnp.int32) for _ in range(NSP)))

    partial = jnp.zeros((L,), jnp.int32)
    for s in range(NSP):
      partial = jnp.where(lane == s, jnp.broadcast_to(jnp.sum(accs[s]), (L,)),
                          partial)
    cm1 = jnp.maximum(c, 1) - 1
    base = plsc.load_gather(cpref_v, [sp_lane * CP + w])
    local = jnp.where(lsub > 0,
                      plsc.load_gather(cum_v, [sp_lane * NSUB + cm1]), 0)
    total = plsc.load_gather(cpref_v, [sp_lane * CP + NW])
    return jnp.where(i >= NS, total, base + local + partial)

  def row_body(r, p_prev):
    p_cur = compute_p(wid * OFF_W + r, r)
    vals = (p_cur - p_prev).astype(jnp.float32)
    base = (r - 1) * NSP
    plsc.store_scatter(outbuf, [jnp.minimum(base + lane, OFF_W * NSP - 1)],
                       vals, mask=lane < NSP)
    return p_cur

  lax.fori_loop(1, OFF_W + 1, row_body, compute_p(wid * OFF_W, 0))
  pltpu.sync_copy(outbuf, out_hbm.at[pl.ds(wid * OFF_W * NSP, OFF_W * NSP)])


_SC_PARAMS = pltpu.CompilerParams(needs_layout_passes=False)

_hist_call = pl.kernel(
    _hist_body,
    out_type=jax.ShapeDtypeStruct((NSP * NSUB,), jnp.int32),
    mesh=_MESH,
    compiler_params=_SC_PARAMS,
    scratch_types=[
        pltpu.VMEM((CHUNK,), jnp.int32),
        pltpu.VMEM((NSP * NSUB_W,), jnp.int32),
        pltpu.SemaphoreType.DMA,
    ],
)

_compose_call = pl.kernel(
    _compose_body,
    out_type=jax.ShapeDtypeStruct((NS * NSP,), jnp.float32),
    mesh=_MESH,
    compiler_params=_SC_PARAMS,
    scratch_types=[
        pltpu.VMEM((NS,), jnp.int32),
        pltpu.VMEM((NSP * NSUB,), jnp.int32),
        pltpu.VMEM((NSP * CP,), jnp.int32),
        pltpu.VMEM((GROWS,), jnp.int32),
        pltpu.VMEM((GROWS, SUB), jnp.int32),
        pltpu.VMEM((OFF_W * NSP,), jnp.float32),
        pltpu.SemaphoreType.DMA,
    ],
)


@jax.jit
def kernel(positions, cells, species, cell_shifts, centers, pairs,
           structure_centers, structure_pairs, structure_offsets):
  cum = _hist_call(species)
  flat = _compose_call(species.reshape(NSUB, SUB), structure_offsets, cum)
  return flat.reshape(NS, NSP)
